# Initial kernel scaffold; baseline (speedup 1.0000x reference)
import jax
import jax.numpy as jnp
from jax.experimental import pallas as pl


def kernel(year, month, day, day_of_week, hour, minute, emb_year, emb_month, emb_day, emb_day_of_week, emb_hour, emb_minute, W1, b1, W2, b2):
    raise NotImplementedError("write your pallas kernel here")


# speedup vs baseline: 10.7356x; 10.7356x over previous
import functools

import jax
import jax.numpy as jnp
from jax.experimental import pallas as pl
from jax.experimental.pallas import tpu as pltpu

_NUM_BINS = 50
_F = 6
_EMB = 64
_H1 = 256
_H2 = 128
_BT = 512


def _mlp_kernel(idx_ref, embs_ref, W1_ref, b1_ref, W2_ref, b2_ref,
                out_ref, T_ref):
    @pl.when(pl.program_id(0) == 0)
    def _():
        for f in range(_F):
            T_ref[f] = jnp.dot(embs_ref[f],
                               W1_ref[f * _EMB:(f + 1) * _EMB, :],
                               preferred_element_type=jnp.float32)

    idx = jax.lax.rem(idx_ref[0], _NUM_BINS)
    acc = None
    for f in range(_F):
        row = idx[f]
        iot = jax.lax.broadcasted_iota(jnp.int32, (_NUM_BINS, _BT), 0)
        ohT = (row[None, :] == iot).astype(jnp.float32)
        part = jax.lax.dot_general(
            ohT, T_ref[f], (((0,), (0,)), ((), ())),
            preferred_element_type=jnp.float32)
        acc = part if acc is None else acc + part
    h1 = acc + b1_ref[...]
    h1 = jnp.where(h1 >= 0, h1, 0.2 * h1)
    h2 = jnp.dot(h1, W2_ref[...], preferred_element_type=jnp.float32)
    h2 = h2 + b2_ref[...]
    out_ref[...] = jnp.where(h2 >= 0, h2, 0.2 * h2)


def kernel(year, month, day, day_of_week, hour, minute,
           emb_year, emb_month, emb_day, emb_day_of_week, emb_hour,
           emb_minute, W1, b1, W2, b2):
    B = year.shape[0]
    grid = B // _BT
    idx = jnp.stack([year, month, day, day_of_week, hour, minute]
                    ).astype(jnp.int32)
    idx = idx.reshape(_F, grid, _BT).transpose(1, 0, 2)
    embs = jnp.stack([emb_year, emb_month, emb_day, emb_day_of_week,
                      emb_hour, emb_minute])
    out = pl.pallas_call(
        _mlp_kernel,
        grid=(grid,),
        in_specs=[
            pl.BlockSpec((1, _F, _BT), lambda i: (i, 0, 0)),
            pl.BlockSpec((_F, _NUM_BINS, _EMB), lambda i: (0, 0, 0)),
            pl.BlockSpec(W1.shape, lambda i: (0, 0)),
            pl.BlockSpec((1, _H1), lambda i: (0, 0)),
            pl.BlockSpec(W2.shape, lambda i: (0, 0)),
            pl.BlockSpec((1, _H2), lambda i: (0, 0)),
        ],
        out_specs=pl.BlockSpec((_BT, _H2), lambda i: (i, 0)),
        out_shape=jax.ShapeDtypeStruct((B, _H2), jnp.float32),
        scratch_shapes=[pltpu.VMEM((_F, _NUM_BINS, _H1), jnp.float32)],
    )(idx, embs, W1, b1.reshape(1, _H1), W2, b2.reshape(1, _H2))
    return out
